# baseline (speedup 1.0000x reference)
import functools

import jax
import jax.numpy as jnp
from jax import lax
from jax.experimental import pallas as pl
from jax.experimental.pallas import tpu as pltpu
from jax.experimental.pallas import tpu_sc as plsc

N_TOTAL = 100000
D = 32
Q = 1024
K = 16
BN = 4096
NB = 25
G = 128
M = BN // G
NG = NB * G
QB = 256
TD = 48
NEG = float(-jnp.inf)
BIG = 2 ** 30


def _phase1_body(x_ref, mu_ref, xsq_ref, msq_ref,
                 f1_ref, f2_ref, f3_ref, c1_ref, c2_ref, c3_ref):
    blk = pl.program_id(0)
    base = blk * BN

    x = x_ref[...]
    mu = mu_ref[...]
    x_sq = xsq_ref[...]
    m_sq = msq_ref[...]
    xm = lax.dot_general(x, mu, (((1,), (1,)), ((), ())),
                         preferred_element_type=jnp.float32)
    d = jnp.sqrt(jnp.maximum(x_sq + m_sq - 2.0 * xm, 0.0))
    iota_b = lax.broadcasted_iota(jnp.int32, (1, BN), 1)
    d = jnp.where(base + iota_b < N_TOTAL, d, NEG)

    f1 = d[:, 0:G]
    m1 = jnp.zeros(f1.shape, jnp.int32)
    f2 = jnp.full(f1.shape, NEG, jnp.float32)
    m2 = jnp.zeros(f1.shape, jnp.int32)
    f3 = jnp.full(f1.shape, NEG, jnp.float32)
    m3 = jnp.zeros(f1.shape, jnp.int32)
    for m in range(1, M):
        s = d[:, m * G:(m + 1) * G]
        mm = jnp.int32(m)
        b1 = s > f1
        b2 = s > f2
        b3 = s > f3
        f3 = jnp.where(b2, f2, jnp.where(b3, s, f3))
        m3 = jnp.where(b2, m2, jnp.where(b3, mm, m3))
        f2 = jnp.where(b1, f1, jnp.where(b2, s, f2))
        m2 = jnp.where(b1, m1, jnp.where(b2, mm, m2))
        f1 = jnp.where(b1, s, f1)
        m1 = jnp.where(b1, mm, m1)

    iota_g = lax.broadcasted_iota(jnp.int32, (1, G), 1)
    f1_ref[...] = f1
    f2_ref[...] = f2
    f3_ref[...] = f3
    c1_ref[...] = base + m1 * G + iota_g
    c2_ref[...] = base + m2 * G + iota_g
    c3_ref[...] = base + m3 * G + iota_g


@jax.jit
def _phase1(x, mu_pad, x_sq, m_sq):
    specs_out = [pl.BlockSpec((Q, G), lambda i: (0, i)) for _ in range(6)]
    return pl.pallas_call(
        _phase1_body,
        grid=(NB,),
        in_specs=[pl.BlockSpec((Q, D), lambda i: (0, 0)),
                  pl.BlockSpec((BN, D), lambda i: (i, 0)),
                  pl.BlockSpec((Q, 1), lambda i: (0, 0)),
                  pl.BlockSpec((1, BN), lambda i: (0, i))],
        out_specs=specs_out,
        out_shape=[jax.ShapeDtypeStruct((Q, NG), jnp.float32),
                   jax.ShapeDtypeStruct((Q, NG), jnp.float32),
                   jax.ShapeDtypeStruct((Q, NG), jnp.float32),
                   jax.ShapeDtypeStruct((Q, NG), jnp.int32),
                   jax.ShapeDtypeStruct((Q, NG), jnp.int32),
                   jax.ShapeDtypeStruct((Q, NG), jnp.int32)],
    )(x, mu_pad, x_sq, m_sq)


def _phase2_body(f1_ref, f2_ref, f3_ref, c1_ref, c2_ref, c3_ref, oidx_ref):
    fs = [f1_ref[...], f2_ref[...], f3_ref[...]]
    cs = [c1_ref[...], c2_ref[...], c3_ref[...]]

    idxs = []
    for _ in range(K):
        m = jnp.max(fs[0], axis=1, keepdims=True)
        for f in fs[1:]:
            m = jnp.maximum(m, jnp.max(f, axis=1, keepdims=True))
        col = jnp.full(m.shape, BIG, jnp.int32)
        for f, c in zip(fs, cs):
            col = jnp.minimum(
                col, jnp.min(jnp.where(f == m, c, BIG), axis=1, keepdims=True))
        fs = [jnp.where((f == m) & (c == col), NEG, f)
              for f, c in zip(fs, cs)]
        idxs.append(col)

    oidx_ref[...] = jnp.concatenate(idxs, axis=1)


@jax.jit
def _phase2(f1, f2, f3, c1, c2, c3):
    return pl.pallas_call(
        _phase2_body,
        grid=(Q // QB,),
        in_specs=[pl.BlockSpec((QB, NG), lambda i: (i, 0)) for _ in range(6)],
        out_specs=pl.BlockSpec((QB, K), lambda i: (i, 0)),
        out_shape=jax.ShapeDtypeStruct((Q, K), jnp.int32),
    )(f1, f2, f3, c1, c2, c3)


_NC, _NS = 2, 16
_NW = _NC * _NS
_BPW = (Q * K) // _NW

AKD = 16


@functools.cache
def _sc_gather_fn():
    mesh = plsc.VectorSubcoreMesh(core_axis_name="c", subcore_axis_name="s")

    @functools.partial(
        pl.kernel, mesh=mesh,
        compiler_params=pltpu.CompilerParams(use_tc_tiling_on_sc=False),
        out_type=[jax.ShapeDtypeStruct((Q * K, D), jnp.float32),
                  jax.ShapeDtypeStruct((Q * K, AKD), jnp.float32)],
        scratch_types=[pltpu.VMEM((_BPW,), jnp.int32),
                       pltpu.VMEM((_BPW, D), jnp.float32),
                       pltpu.VMEM((_BPW, AKD), jnp.float32),
                       pltpu.SemaphoreType.DMA,
                       pltpu.SemaphoreType.DMA],
    )
    def _sc_gather(mu_hbm, ak_hbm, idx_hbm, omu_hbm, oak_hbm,
                   idx_v, mu_v, ak_v, sem1, sem2):
        wid = lax.axis_index("s") * _NC + lax.axis_index("c")
        base = wid * _BPW
        pltpu.sync_copy(idx_hbm.at[pl.ds(base, _BPW)], idx_v)
        cp1 = pltpu.async_copy(mu_hbm.at[idx_v], mu_v, sem1)
        cp2 = pltpu.async_copy(ak_hbm.at[idx_v], ak_v, sem2)
        cp1.wait()
        cp2.wait()
        pltpu.sync_copy(mu_v, omu_hbm.at[pl.ds(base, _BPW)])
        pltpu.sync_copy(ak_v, oak_hbm.at[pl.ds(base, _BPW)])

    return _sc_gather


def kernel(x, mu, alpha, kappa, k):
    mu_pad = jnp.concatenate(
        [mu, jnp.zeros((NB * BN - N_TOTAL, D), mu.dtype)], axis=0)
    x_sq = jnp.sum(x * x, axis=-1, keepdims=True)
    m_sq_pad = jnp.sum(mu_pad * mu_pad, axis=-1)[None, :]
    f1, f2, f3, c1, c2, c3 = _phase1(x, mu_pad, x_sq, m_sq_pad)
    topk_idx = _phase2(f1, f2, f3, c1, c2, c3)
    idx = topk_idx + (jnp.asarray(k, topk_idx.dtype) - K)
    ak = jnp.concatenate(
        [alpha[:, None], kappa[:, None],
         jnp.zeros((N_TOTAL, AKD - 2), jnp.float32)], axis=1)
    g_mu, g_ak = _sc_gather_fn()(mu, ak, idx.reshape(-1))
    neighbors_mu = g_mu.reshape(Q, K, D)
    neighbors_alpha = g_ak[:, 0].reshape(Q, K)
    neighbors_kappa = g_ak[:, 1].reshape(Q, K)
    return (neighbors_mu, neighbors_alpha, neighbors_kappa)


# speedup vs baseline: 1.0454x; 1.0454x over previous
import functools

import jax
import jax.numpy as jnp
from jax import lax
from jax.experimental import pallas as pl
from jax.experimental.pallas import tpu as pltpu
from jax.experimental.pallas import tpu_sc as plsc

N_TOTAL = 100000
D = 32
Q = 1024
K = 16
BN = 4096
NB = 25
G = 128
M = BN // G
NG = NB * G
QB = 256
TD = 48
NEG = float(-jnp.inf)
BIG = 2 ** 30


def _phase1_body(x_ref, mu_ref, xsq_ref, msq_ref,
                 f1_ref, f2_ref, f3_ref, c1_ref, c2_ref, c3_ref):
    blk = pl.program_id(0)
    base = blk * BN

    x = x_ref[...]
    mu = mu_ref[...]
    x_sq = xsq_ref[...]
    m_sq = msq_ref[...]
    xm = lax.dot_general(x, mu, (((1,), (1,)), ((), ())),
                         preferred_element_type=jnp.float32)
    d = jnp.sqrt(jnp.maximum(x_sq + m_sq - 2.0 * xm, 0.0))

    f1 = d[:, 0:G]
    m1 = jnp.zeros(f1.shape, jnp.int32)
    f2 = jnp.full(f1.shape, NEG, jnp.float32)
    m2 = jnp.zeros(f1.shape, jnp.int32)
    f3 = jnp.full(f1.shape, NEG, jnp.float32)
    m3 = jnp.zeros(f1.shape, jnp.int32)
    for m in range(1, M):
        s = d[:, m * G:(m + 1) * G]
        mm = jnp.int32(m)
        b1 = s > f1
        b2 = s > f2
        b3 = s > f3
        f3 = jnp.where(b2, f2, jnp.where(b3, s, f3))
        m3 = jnp.where(b2, m2, jnp.where(b3, mm, m3))
        f2 = jnp.where(b1, f1, jnp.where(b2, s, f2))
        m2 = jnp.where(b1, m1, jnp.where(b2, mm, m2))
        f1 = jnp.where(b1, s, f1)
        m1 = jnp.where(b1, mm, m1)

    iota_g = lax.broadcasted_iota(jnp.int32, (1, G), 1)
    f1_ref[...] = f1
    f2_ref[...] = f2
    f3_ref[...] = f3
    c1_ref[...] = base + m1 * G + iota_g
    c2_ref[...] = base + m2 * G + iota_g
    c3_ref[...] = base + m3 * G + iota_g


@jax.jit
def _phase1(x, mu_pad, x_sq, m_sq):
    specs_out = [pl.BlockSpec((Q, G), lambda i: (0, i)) for _ in range(6)]
    return pl.pallas_call(
        _phase1_body,
        grid=(NB,),
        in_specs=[pl.BlockSpec((Q, D), lambda i: (0, 0)),
                  pl.BlockSpec((BN, D), lambda i: (i, 0)),
                  pl.BlockSpec((Q, 1), lambda i: (0, 0)),
                  pl.BlockSpec((1, BN), lambda i: (0, i))],
        out_specs=specs_out,
        out_shape=[jax.ShapeDtypeStruct((Q, NG), jnp.float32),
                   jax.ShapeDtypeStruct((Q, NG), jnp.float32),
                   jax.ShapeDtypeStruct((Q, NG), jnp.float32),
                   jax.ShapeDtypeStruct((Q, NG), jnp.int32),
                   jax.ShapeDtypeStruct((Q, NG), jnp.int32),
                   jax.ShapeDtypeStruct((Q, NG), jnp.int32)],
    )(x, mu_pad, x_sq, m_sq)


def _phase2_body(f1_ref, f2_ref, f3_ref, c1_ref, c2_ref, c3_ref, oidx_ref):
    fs = [f1_ref[...], f2_ref[...], f3_ref[...]]
    cs = [c1_ref[...], c2_ref[...], c3_ref[...]]

    idxs = []
    for _ in range(K):
        m = jnp.max(fs[0], axis=1, keepdims=True)
        for f in fs[1:]:
            m = jnp.maximum(m, jnp.max(f, axis=1, keepdims=True))
        col = jnp.full(m.shape, BIG, jnp.int32)
        for f, c in zip(fs, cs):
            col = jnp.minimum(
                col, jnp.min(jnp.where(f == m, c, BIG), axis=1, keepdims=True))
        fs = [jnp.where(c == col, NEG, f) for f, c in zip(fs, cs)]
        idxs.append(col)

    oidx_ref[...] = jnp.concatenate(idxs, axis=1)


@jax.jit
def _phase2(f1, f2, f3, c1, c2, c3):
    return pl.pallas_call(
        _phase2_body,
        grid=(Q // QB,),
        in_specs=[pl.BlockSpec((QB, NG), lambda i: (i, 0)) for _ in range(6)],
        out_specs=pl.BlockSpec((QB, K), lambda i: (i, 0)),
        out_shape=jax.ShapeDtypeStruct((Q, K), jnp.int32),
    )(f1, f2, f3, c1, c2, c3)


_NC, _NS = 2, 16
_NW = _NC * _NS
_BPW = (Q * K) // _NW

@functools.cache
def _sc_gather_fn():
    mesh = plsc.VectorSubcoreMesh(core_axis_name="c", subcore_axis_name="s")

    @functools.partial(
        pl.kernel, mesh=mesh,
        compiler_params=pltpu.CompilerParams(use_tc_tiling_on_sc=False),
        out_type=jax.ShapeDtypeStruct((Q * K, TD), jnp.float32),
        scratch_types=[pltpu.VMEM((_BPW,), jnp.int32),
                       pltpu.VMEM((_BPW, TD), jnp.float32),
                       pltpu.SemaphoreType.DMA],
    )
    def _sc_gather(table_hbm, idx_hbm, out_hbm, idx_v, rows_v, sem):
        wid = lax.axis_index("s") * _NC + lax.axis_index("c")
        base = wid * _BPW
        pltpu.sync_copy(idx_hbm.at[pl.ds(base, _BPW)], idx_v)
        pltpu.async_copy(table_hbm.at[idx_v], rows_v, sem).wait()
        pltpu.sync_copy(rows_v, out_hbm.at[pl.ds(base, _BPW)])

    return _sc_gather


def kernel(x, mu, alpha, kappa, k):
    mu_pad = jnp.concatenate(
        [mu, jnp.zeros((NB * BN - N_TOTAL, D), mu.dtype)], axis=0)
    x_sq = jnp.sum(x * x, axis=-1, keepdims=True)
    m_sq_pad = jnp.sum(mu_pad * mu_pad, axis=-1)[None, :]
    m_sq_pad = jnp.where(
        jnp.arange(NB * BN)[None, :] < N_TOTAL, m_sq_pad, NEG)
    f1, f2, f3, c1, c2, c3 = _phase1(x, mu_pad, x_sq, m_sq_pad)
    topk_idx = _phase2(f1, f2, f3, c1, c2, c3)
    idx = topk_idx + (jnp.asarray(k, topk_idx.dtype) - K)
    table = jnp.concatenate(
        [mu, alpha[:, None], kappa[:, None],
         jnp.zeros((N_TOTAL, TD - D - 2), jnp.float32)], axis=1)
    g = _sc_gather_fn()(table, idx.reshape(-1))
    neighbors_mu = g[:, :D].reshape(Q, K, D)
    neighbors_alpha = g[:, D].reshape(Q, K)
    neighbors_kappa = g[:, D + 1].reshape(Q, K)
    return (neighbors_mu, neighbors_alpha, neighbors_kappa)
